# baseline (speedup 1.0000x reference)
import functools
import math

import numpy as np
import jax
import jax.numpy as jnp
from jax import lax
from jax.experimental import pallas as pl
from jax.experimental.pallas import tpu as pltpu
from jax.experimental.pallas import tpu_sc as plsc

_EPS = 1e-12
_NSW = 4
_NSH = 3


def _sin_tables(s, h):
    pos = np.arange(s, dtype=np.float32)[:, None]
    pe = np.zeros((s, h), np.float32)
    div = np.exp(np.arange(0, h, 2, dtype=np.float32) * -(math.log(10000.0) / h))
    pe[:, 0::2] = np.sin(pos * div)
    pe[:, 1::2] = np.cos(pos * div)
    hdim = h // 3
    hs = np.zeros((s, hdim), np.float32)
    divh = np.exp(np.arange(0, hdim, 2, dtype=np.float32) * -(math.log(10000.0) / hdim))
    hs[:, 0::2] = np.sin(pos * divh)
    hs[:, 1::2] = np.cos(pos * divh)
    return pe, hs


@functools.lru_cache(maxsize=None)
def _make_sc_gather(T, H, HS, C):
    info = plsc.get_sparse_core_info()
    NC, NS, L = info.num_cores, info.num_subcores, info.num_lanes
    NW = NC * NS
    TPW = T // NW
    NCH = TPW // C
    assert T % NW == 0 and TPW % C == 0 and NCH >= _NSW

    mesh = plsc.VectorSubcoreMesh(core_axis_name="c", subcore_axis_name="s")

    @functools.partial(
        pl.kernel,
        mesh=mesh,
        out_type=(jax.ShapeDtypeStruct((T, H), jnp.float32),
                  jax.ShapeDtypeStruct((T, HS), jnp.float32)),
        scratch_types=[
            pltpu.VMEM((TPW,), jnp.int32),
            pltpu.VMEM((TPW,), jnp.int32),
            pltpu.VMEM((_NSW, C, H), jnp.float32),
            pltpu.VMEM((_NSH, C, HS), jnp.float32),
            pltpu.SemaphoreType.DMA,
            pltpu.SemaphoreType.DMA,
            pltpu.SemaphoreType.DMA,
            pltpu.SemaphoreType.DMA,
        ],
    )
    def k(ids_h, para_h, wtab_h, hs_h, wout_h, hout_h,
          ids_v, para_v, wbuf, hbuf, gsem, gsem2, osem, osem2):
        wid = lax.axis_index("s") * NC + lax.axis_index("c")
        t0 = wid * TPW
        pltpu.sync_copy(ids_h.at[pl.ds(t0, TPW)], ids_v)
        pltpu.sync_copy(para_h.at[pl.ds(t0, TPW)], para_v)

        def wait_wout(c):
            pltpu.make_async_copy(
                wbuf.at[lax.rem(c, _NSW)], wout_h.at[pl.ds(t0 + c * C, C)],
                osem).wait()

        def wait_hout(c):
            pltpu.make_async_copy(
                hbuf.at[lax.rem(c, _NSH)], hout_h.at[pl.ds(t0 + c * C, C)],
                osem2).wait()

        pltpu.async_copy(wtab_h.at[ids_v.at[pl.ds(0, C)]], wbuf.at[0], gsem)
        pltpu.async_copy(hs_h.at[para_v.at[pl.ds(0, C)]], hbuf.at[0], gsem2)

        def chunk_body(c, carry):
            wslot = lax.rem(c, _NSW)
            hslot = lax.rem(c, _NSH)

            @pl.when(c + 1 < NCH)
            def _():
                @pl.when(c + 1 >= _NSW)
                def _():
                    wait_wout(c + 1 - _NSW)

                pltpu.async_copy(
                    wtab_h.at[ids_v.at[pl.ds((c + 1) * C, C)]],
                    wbuf.at[lax.rem(c + 1, _NSW)], gsem)

                @pl.when(c + 1 >= _NSH)
                def _():
                    wait_hout(c + 1 - _NSH)

                pltpu.async_copy(
                    hs_h.at[para_v.at[pl.ds((c + 1) * C, C)]],
                    hbuf.at[lax.rem(c + 1, _NSH)], gsem2)

            pltpu.make_async_copy(
                wtab_h.at[ids_v.at[pl.ds(c * C, C)]], wbuf.at[wslot], gsem).wait()
            pltpu.make_async_copy(
                hs_h.at[para_v.at[pl.ds(c * C, C)]], hbuf.at[hslot], gsem2).wait()
            pltpu.async_copy(wbuf.at[wslot], wout_h.at[pl.ds(t0 + c * C, C)], osem)
            pltpu.async_copy(hbuf.at[hslot], hout_h.at[pl.ds(t0 + c * C, C)], osem2)
            return carry

        lax.fori_loop(0, NCH, chunk_body, 0)
        for c in range(NCH - min(_NSW, NCH), NCH):
            wait_wout(c)
        for c in range(NCH - min(_NSH, NCH), NCH):
            wait_hout(c)

    return k


def _tc_body(wref, hsref, peref, ttref, ttabref, gamref, betref, oref):
    w = wref[...]
    hs = hsref[...]
    pe = peref[...]
    tf = jnp.transpose(ttref[0])
    t0 = ttabref[0:1, :]
    td = ttabref[1:2, :] - t0
    acc = w + pe + (t0 + tf * td) + jnp.concatenate([hs, hs, hs], axis=1)
    mean = jnp.mean(acc, axis=1, keepdims=True)
    cen = acc - mean
    var = jnp.mean(cen * cen, axis=1, keepdims=True)
    inv = lax.rsqrt(var + _EPS)
    oref[...] = (cen * inv) * gamref[...] + betref[...]


@functools.lru_cache(maxsize=None)
def _make_tc_ln(B, S, H, BT):
    T = B * S
    SB = S // BT
    grid = (SB, B)

    return pl.pallas_call(
        _tc_body,
        grid=grid,
        in_specs=[
            pl.BlockSpec((BT, H), lambda sb, b: (b * SB + sb, 0)),
            pl.BlockSpec((BT, H // 3), lambda sb, b: (b * SB + sb, 0)),
            pl.BlockSpec((BT, H), lambda sb, b: (sb, 0)),
            pl.BlockSpec((1, 1, BT), lambda sb, b: (b * SB + sb, 0, 0)),
            pl.BlockSpec((2, H), lambda sb, b: (0, 0)),
            pl.BlockSpec((1, H), lambda sb, b: (0, 0)),
            pl.BlockSpec((1, H), lambda sb, b: (0, 0)),
        ],
        out_specs=pl.BlockSpec((BT, H), lambda sb, b: (b * SB + sb, 0)),
        out_shape=jax.ShapeDtypeStruct((T, H), jnp.float32),
    )


def kernel(input_ids, tok_struct_vec, sent_struct_vec, token_type_ids,
           word_table, type_table, ln_gamma, ln_beta):
    B, S = input_ids.shape
    H = word_table.shape[1]
    BT = 2048
    pe_np, hs_np = _sin_tables(S, H)
    ids = input_ids.reshape(-1).astype(jnp.int32)
    para = tok_struct_vec[..., 0].reshape(-1).astype(jnp.int32)
    tt3 = token_type_ids.reshape(-1, 1, BT).astype(jnp.float32)

    sc = _make_sc_gather(B * S, H, H // 3, 32)
    wrows, hsrows = sc(ids, para, word_table.astype(jnp.float32),
                       jnp.asarray(hs_np))

    tc = _make_tc_ln(B, S, H, BT)
    out = tc(wrows, hsrows, jnp.asarray(pe_np), tt3,
             type_table.astype(jnp.float32),
             ln_gamma.reshape(1, H).astype(jnp.float32),
             ln_beta.reshape(1, H).astype(jnp.float32))
    return out.reshape(B, S, H)


# speedup vs baseline: 1.0022x; 1.0022x over previous
import functools
import math

import numpy as np
import jax
import jax.numpy as jnp
from jax import lax
from jax.experimental import pallas as pl
from jax.experimental.pallas import tpu as pltpu
from jax.experimental.pallas import tpu_sc as plsc

_EPS = 1e-12
_NSLOT = 3


def _sin_tables(s, h):
    pos = np.arange(s, dtype=np.float32)[:, None]
    pe = np.zeros((s, h), np.float32)
    div = np.exp(np.arange(0, h, 2, dtype=np.float32) * -(math.log(10000.0) / h))
    pe[:, 0::2] = np.sin(pos * div)
    pe[:, 1::2] = np.cos(pos * div)
    hdim = h // 3
    hs = np.zeros((s, hdim), np.float32)
    divh = np.exp(np.arange(0, hdim, 2, dtype=np.float32) * -(math.log(10000.0) / hdim))
    hs[:, 0::2] = np.sin(pos * divh)
    hs[:, 1::2] = np.cos(pos * divh)
    return pe, hs


@functools.lru_cache(maxsize=None)
def _make_sc_gather(T, H, HS, C):
    info = plsc.get_sparse_core_info()
    NC, NS, L = info.num_cores, info.num_subcores, info.num_lanes
    NW = NC * NS
    TPW = T // NW
    NCH = TPW // C
    assert T % NW == 0 and TPW % C == 0 and NCH >= _NSLOT

    mesh = plsc.VectorSubcoreMesh(core_axis_name="c", subcore_axis_name="s")

    @functools.partial(
        pl.kernel,
        mesh=mesh,
        out_type=(jax.ShapeDtypeStruct((T, H), jnp.float32),
                  jax.ShapeDtypeStruct((T, HS), jnp.float32)),
        scratch_types=[
            pltpu.VMEM((TPW,), jnp.int32),
            pltpu.VMEM((TPW,), jnp.int32),
            pltpu.VMEM((_NSLOT, C, H), jnp.float32),
            pltpu.VMEM((_NSLOT, C, HS), jnp.float32),
            pltpu.SemaphoreType.DMA,
            pltpu.SemaphoreType.DMA,
            pltpu.SemaphoreType.DMA,
            pltpu.SemaphoreType.DMA,
        ],
    )
    def k(ids_h, para_h, wtab_h, hs_h, wout_h, hout_h,
          ids_v, para_v, wbuf, hbuf, gsem, gsem2, osem, osem2):
        wid = lax.axis_index("s") * NC + lax.axis_index("c")
        t0 = wid * TPW
        pltpu.sync_copy(ids_h.at[pl.ds(t0, TPW)], ids_v)
        pltpu.sync_copy(para_h.at[pl.ds(t0, TPW)], para_v)

        def start_gather(c, slot):
            pltpu.async_copy(
                wtab_h.at[ids_v.at[pl.ds(c * C, C)]], wbuf.at[slot], gsem)
            pltpu.async_copy(
                hs_h.at[para_v.at[pl.ds(c * C, C)]], hbuf.at[slot], gsem2)

        def wait_writeout(c, slot):
            pltpu.make_async_copy(
                wbuf.at[slot], wout_h.at[pl.ds(t0 + c * C, C)], osem).wait()
            pltpu.make_async_copy(
                hbuf.at[slot], hout_h.at[pl.ds(t0 + c * C, C)], osem2).wait()

        start_gather(0, 0)

        def chunk_body(c, carry):
            slot = lax.rem(c, _NSLOT)

            @pl.when(c + 1 < NCH)
            def _():
                nslot = lax.rem(c + 1, _NSLOT)

                @pl.when(c + 1 >= _NSLOT)
                def _():
                    wait_writeout(c + 1 - _NSLOT, nslot)

                start_gather(c + 1, nslot)

            pltpu.make_async_copy(
                wtab_h.at[ids_v.at[pl.ds(c * C, C)]], wbuf.at[slot], gsem).wait()
            pltpu.make_async_copy(
                hs_h.at[para_v.at[pl.ds(c * C, C)]], hbuf.at[slot], gsem2).wait()
            pltpu.async_copy(wbuf.at[slot], wout_h.at[pl.ds(t0 + c * C, C)], osem)
            pltpu.async_copy(hbuf.at[slot], hout_h.at[pl.ds(t0 + c * C, C)], osem2)
            return carry

        lax.fori_loop(0, NCH, chunk_body, 0)
        for c in range(NCH - min(_NSLOT, NCH), NCH):
            wait_writeout(c, c % _NSLOT)

    return k


def _tc_body(wref, hsref, peref, ttref, ttabref, gamref, betref, oref):
    w = wref[...]
    hs = hsref[...]
    pe = peref[...]
    tf = jnp.transpose(ttref[0])
    t0 = ttabref[0:1, :]
    td = ttabref[1:2, :] - t0
    acc = w + pe + (t0 + tf * td) + jnp.concatenate([hs, hs, hs], axis=1)
    mean = jnp.mean(acc, axis=1, keepdims=True)
    cen = acc - mean
    var = jnp.mean(cen * cen, axis=1, keepdims=True)
    inv = lax.rsqrt(var + _EPS)
    oref[...] = (cen * inv) * gamref[...] + betref[...]


@functools.lru_cache(maxsize=None)
def _make_tc_ln(B, S, H, BT):
    T = B * S
    SB = S // BT
    grid = (SB, B)

    return pl.pallas_call(
        _tc_body,
        grid=grid,
        in_specs=[
            pl.BlockSpec((BT, H), lambda sb, b: (b * SB + sb, 0)),
            pl.BlockSpec((BT, H // 3), lambda sb, b: (b * SB + sb, 0)),
            pl.BlockSpec((BT, H), lambda sb, b: (sb, 0)),
            pl.BlockSpec((1, 1, BT), lambda sb, b: (b * SB + sb, 0, 0)),
            pl.BlockSpec((2, H), lambda sb, b: (0, 0)),
            pl.BlockSpec((1, H), lambda sb, b: (0, 0)),
            pl.BlockSpec((1, H), lambda sb, b: (0, 0)),
        ],
        out_specs=pl.BlockSpec((BT, H), lambda sb, b: (b * SB + sb, 0)),
        out_shape=jax.ShapeDtypeStruct((T, H), jnp.float32),
    )


def kernel(input_ids, tok_struct_vec, sent_struct_vec, token_type_ids,
           word_table, type_table, ln_gamma, ln_beta):
    B, S = input_ids.shape
    H = word_table.shape[1]
    BT = 2048
    pe_np, hs_np = _sin_tables(S, H)
    ids = input_ids.reshape(-1).astype(jnp.int32)
    para = tok_struct_vec[..., 0].reshape(-1).astype(jnp.int32)
    tt3 = token_type_ids.reshape(-1, 1, BT).astype(jnp.float32)

    sc = _make_sc_gather(B * S, H, H // 3, 32)
    wrows, hsrows = sc(ids, para, word_table.astype(jnp.float32),
                       jnp.asarray(hs_np))

    tc = _make_tc_ln(B, S, H, BT)
    out = tc(wrows, hsrows, jnp.asarray(pe_np), tt3,
             type_table.astype(jnp.float32),
             ln_gamma.reshape(1, H).astype(jnp.float32),
             ln_beta.reshape(1, H).astype(jnp.float32))
    return out.reshape(B, S, H)
